# baseline (speedup 1.0000x reference)
import functools

import jax
import jax.numpy as jnp
from jax import lax
from jax.experimental import pallas as pl
from jax.experimental.pallas import tpu as pltpu
from jax.experimental.pallas import tpu_sc as plsc

D_MODEL = 1024
EPS = 1e-5

_NUM_WORKERS = 32
_NCHUNKS = 2


def _sc_gather_chunk(token_table, ids_2d, k, cs, ch, nb):
    b, s = ids_2d.shape
    d = token_table.shape[1]
    n = b * cs
    b_per_w = n // _NUM_WORKERS
    w_per_batch = _NUM_WORKERS // b
    n_ch = b_per_w // ch
    mesh = plsc.VectorSubcoreMesh(core_axis_name="c", subcore_axis_name="s")

    @functools.partial(
        pl.kernel,
        mesh=mesh,
        out_type=jax.ShapeDtypeStruct((n, d), jnp.float32),
        scratch_types=[
            pltpu.VMEM((b_per_w,), jnp.int32),
        ]
        + [pltpu.VMEM((ch, d), jnp.float32) for _ in range(nb)]
        + [pltpu.SemaphoreType.DMA for _ in range(2 * nb)],
    )
    def gather_kernel(table_hbm, idx_hbm, out_hbm, idx_v, *rest):
        bufs = rest[:nb]
        gsems = rest[nb:2 * nb]
        wsems = rest[2 * nb:]
        wid = lax.axis_index("s") * 2 + lax.axis_index("c")
        bb = wid // w_per_batch
        col0 = k * cs + (wid % w_per_batch) * b_per_w
        base = wid * b_per_w
        pltpu.sync_copy(idx_hbm.at[bb, pl.ds(col0, b_per_w)], idx_v)

        def start_gather(c):
            cp = pltpu.make_async_copy(
                table_hbm.at[idx_v.at[pl.ds(c * ch, ch)]],
                bufs[c % nb],
                gsems[c % nb],
            )
            cp.start()
            return cp

        def start_wb(c):
            cp = pltpu.make_async_copy(
                bufs[c % nb],
                out_hbm.at[pl.ds(base + c * ch, ch)],
                wsems[c % nb],
            )
            cp.start()
            return cp

        gathers = [None] * n_ch
        wbs = [None] * n_ch
        for c in range(min(nb - 1, n_ch)):
            gathers[c] = start_gather(c)
        for c in range(n_ch):
            gathers[c].wait()
            wbs[c] = start_wb(c)
            j = c + nb - 1
            if j < n_ch:
                if c >= 1:
                    wbs[c - 1].wait()
                gathers[j] = start_gather(j)
        for c in range(max(0, n_ch - nb), n_ch):
            if wbs[c] is not None:
                wbs[c].wait()

    return gather_kernel(token_table, ids_2d)


def _ln_math(x, segtab_ref, gamma_ref, beta_ref, segf):
    s0 = segtab_ref[0, :]
    s1 = segtab_ref[1, :]
    x = x + s0[None, :] + segf * (s1 - s0)[None, :]
    mean = jnp.mean(x, axis=-1, keepdims=True)
    xc = x - mean
    var = jnp.mean(xc * xc, axis=-1, keepdims=True)
    return gamma_ref[...] * (xc * lax.rsqrt(var + EPS)) + beta_ref[...]


def _ln_body(g_ref, pos_ref, segf_ref, segtab_ref, gamma_ref, beta_ref, o_ref):
    x = g_ref[...] + pos_ref[...]
    o_ref[...] = _ln_math(x, segtab_ref, gamma_ref, beta_ref, segf_ref[...])


def _ln_body_acc(g_ref, pos_ref, segf_ref, segtab_ref, gamma_ref, beta_ref,
                 prev_ref, o_ref):
    del prev_ref
    x = g_ref[...] + pos_ref[...]
    o_ref[...] = _ln_math(x, segtab_ref, gamma_ref, beta_ref, segf_ref[...])


def _tc_ln_chunk(g_k, pos_table, segf, seg_table, gamma, beta, prev, off,
                 cs, batches, s_total, blk):
    d = g_k.shape[1]
    n_total = batches * s_total
    jb = cs // blk
    sb = s_total // blk
    ob = off // blk
    in_specs = [
        pl.BlockSpec((blk, d), lambda jj, bb: (bb * jb + jj, 0)),
        pl.BlockSpec((blk, d), lambda jj, bb: (ob + jj, 0)),
        pl.BlockSpec((blk, 1), lambda jj, bb: (bb * sb + ob + jj, 0)),
        pl.BlockSpec((2, d), lambda jj, bb: (0, 0)),
        pl.BlockSpec((1, d), lambda jj, bb: (0, 0)),
        pl.BlockSpec((1, d), lambda jj, bb: (0, 0)),
    ]
    args = [g_k, pos_table, segf, seg_table, gamma, beta]
    if prev is None:
        body = _ln_body
        aliases = {}
    else:
        body = _ln_body_acc
        in_specs = in_specs + [pl.BlockSpec(memory_space=pltpu.MemorySpace.HBM)]
        args = args + [prev]
        aliases = {6: 0}
    return pl.pallas_call(
        body,
        grid=(jb, batches),
        in_specs=in_specs,
        out_specs=pl.BlockSpec((blk, d), lambda jj, bb: (bb * sb + ob + jj, 0)),
        out_shape=jax.ShapeDtypeStruct((n_total, d), jnp.float32),
        input_output_aliases=aliases,
    )(*args)


def kernel(input_ids, segment_ids, token_table, pos_table, seg_table, gamma, beta):
    b, s = input_ids.shape
    cs = s // _NCHUNKS
    ids32 = input_ids.astype(jnp.int32)
    segf = segment_ids.reshape(-1, 1).astype(jnp.float32)
    gamma2 = gamma.reshape(1, -1)
    beta2 = beta.reshape(1, -1)

    gathered = [
        _sc_gather_chunk(token_table, ids32, k, cs, ch=16, nb=7)
        for k in range(_NCHUNKS)
    ]

    out = None
    for k in range(_NCHUNKS):
        out = _tc_ln_chunk(
            gathered[k], pos_table, segf, seg_table, gamma2, beta2,
            out, k * cs, cs, b, s, blk=256,
        )
    return out.reshape(b, s, D_MODEL)


# speedup vs baseline: 1.1262x; 1.1262x over previous
import functools

import jax
import jax.numpy as jnp
from jax import lax
from jax.experimental import pallas as pl
from jax.experimental.pallas import tpu as pltpu
from jax.experimental.pallas import tpu_sc as plsc

D_MODEL = 1024
EPS = 1e-5

_NUM_WORKERS = 32
_NCHUNKS = 2


def _sc_gather_chunk(token_table, ids_2d, k, cs, ch, nb):
    b, s = ids_2d.shape
    d = token_table.shape[1]
    n = b * cs
    b_per_w = n // _NUM_WORKERS
    w_per_batch = _NUM_WORKERS // b
    n_ch = b_per_w // ch
    mesh = plsc.VectorSubcoreMesh(core_axis_name="c", subcore_axis_name="s")

    @functools.partial(
        pl.kernel,
        mesh=mesh,
        out_type=jax.ShapeDtypeStruct((n, d), jnp.float32),
        scratch_types=[
            pltpu.VMEM((b_per_w,), jnp.int32),
        ]
        + [pltpu.VMEM((ch, d), jnp.float32) for _ in range(nb)]
        + [pltpu.SemaphoreType.DMA for _ in range(2 * nb)],
    )
    def gather_kernel(table_hbm, idx_hbm, out_hbm, idx_v, *rest):
        bufs = rest[:nb]
        gsems = rest[nb:2 * nb]
        wsems = rest[2 * nb:]
        wid = lax.axis_index("s") * 2 + lax.axis_index("c")
        bb = wid // w_per_batch
        col0 = k * cs + (wid % w_per_batch) * b_per_w
        base = wid * b_per_w
        pltpu.sync_copy(idx_hbm.at[bb, pl.ds(col0, b_per_w)], idx_v)

        def start_gather(c):
            cp = pltpu.make_async_copy(
                table_hbm.at[idx_v.at[pl.ds(c * ch, ch)]],
                bufs[c % nb],
                gsems[c % nb],
            )
            cp.start()
            return cp

        def start_wb(c):
            cp = pltpu.make_async_copy(
                bufs[c % nb],
                out_hbm.at[pl.ds(base + c * ch, ch)],
                wsems[c % nb],
            )
            cp.start()
            return cp

        gathers = [None] * n_ch
        wbs = [None] * n_ch
        for c in range(min(nb - 1, n_ch)):
            gathers[c] = start_gather(c)
        for c in range(n_ch):
            gathers[c].wait()
            wbs[c] = start_wb(c)
            j = c + nb - 1
            if j < n_ch:
                if c >= 1:
                    wbs[c - 1].wait()
                gathers[j] = start_gather(j)
        for c in range(max(0, n_ch - nb), n_ch):
            if wbs[c] is not None:
                wbs[c].wait()

    return gather_kernel(token_table, ids_2d)


def _ln_math(x, segtab_ref, gamma_ref, beta_ref, segf):
    s0 = segtab_ref[0, :]
    s1 = segtab_ref[1, :]
    x = x + s0[None, :] + segf * (s1 - s0)[None, :]
    mean = jnp.mean(x, axis=-1, keepdims=True)
    xc = x - mean
    var = jnp.mean(xc * xc, axis=-1, keepdims=True)
    return gamma_ref[...] * (xc * lax.rsqrt(var + EPS)) + beta_ref[...]


def _ln_body(g_ref, pos_ref, segf_ref, segtab_ref, gamma_ref, beta_ref, o_ref):
    x = g_ref[...] + pos_ref[...]
    o_ref[...] = _ln_math(x, segtab_ref, gamma_ref, beta_ref, segf_ref[...])


def _ln_body_acc(g_ref, pos_ref, segf_ref, segtab_ref, gamma_ref, beta_ref,
                 prev_ref, o_ref):
    del prev_ref
    x = g_ref[...] + pos_ref[...]
    o_ref[...] = _ln_math(x, segtab_ref, gamma_ref, beta_ref, segf_ref[...])


def _tc_ln_chunk(g_k, pos_table, segf, seg_table, gamma, beta, prev, off,
                 cs, batches, s_total, blk):
    d = g_k.shape[1]
    n_total = batches * s_total
    sb = s_total // blk
    ob = off // blk
    in_specs = [
        pl.BlockSpec((blk, d), lambda bb: (bb, 0)),
        pl.BlockSpec((blk, d), lambda bb: (ob, 0)),
        pl.BlockSpec((blk, 1), lambda bb: (bb * sb + ob, 0)),
        pl.BlockSpec((2, d), lambda bb: (0, 0)),
        pl.BlockSpec((1, d), lambda bb: (0, 0)),
        pl.BlockSpec((1, d), lambda bb: (0, 0)),
    ]
    args = [g_k, pos_table, segf, seg_table, gamma, beta]
    if prev is None:
        body = _ln_body
        aliases = {}
    else:
        body = _ln_body_acc
        in_specs = in_specs + [pl.BlockSpec(memory_space=pltpu.MemorySpace.HBM)]
        args = args + [prev]
        aliases = {6: 0}
    return pl.pallas_call(
        body,
        grid=(batches,),
        in_specs=in_specs,
        out_specs=pl.BlockSpec((blk, d), lambda bb: (bb * sb + ob, 0)),
        out_shape=jax.ShapeDtypeStruct((n_total, d), jnp.float32),
        input_output_aliases=aliases,
    )(*args)


def kernel(input_ids, segment_ids, token_table, pos_table, seg_table, gamma, beta):
    b, s = input_ids.shape
    cs = s // _NCHUNKS
    ids32 = input_ids.astype(jnp.int32)
    segf = segment_ids.reshape(-1, 1).astype(jnp.float32)
    gamma2 = gamma.reshape(1, -1)
    beta2 = beta.reshape(1, -1)

    gathered = [
        _sc_gather_chunk(token_table, ids32, k, cs, ch=16, nb=7)
        for k in range(_NCHUNKS)
    ]

    out = None
    for k in range(_NCHUNKS):
        out = _tc_ln_chunk(
            gathered[k], pos_table, segf, seg_table, gamma2, beta2,
            out, k * cs, cs, b, s, blk=cs,
        )
    return out.reshape(b, s, D_MODEL)


# speedup vs baseline: 1.1464x; 1.0179x over previous
import functools

import jax
import jax.numpy as jnp
from jax import lax
from jax.experimental import pallas as pl
from jax.experimental.pallas import tpu as pltpu
from jax.experimental.pallas import tpu_sc as plsc

D_MODEL = 1024
EPS = 1e-5

_NUM_WORKERS = 32
_NCHUNKS = 2


def _sc_gather_chunk(token_table, ids_2d, k, cs, ch, nb):
    b, s = ids_2d.shape
    d = token_table.shape[1]
    n = b * cs
    b_per_w = n // _NUM_WORKERS
    w_per_batch = _NUM_WORKERS // b
    n_ch = b_per_w // ch
    mesh = plsc.VectorSubcoreMesh(core_axis_name="c", subcore_axis_name="s")

    @functools.partial(
        pl.kernel,
        mesh=mesh,
        out_type=jax.ShapeDtypeStruct((n, d), jnp.float32),
        scratch_types=[
            pltpu.VMEM((b_per_w,), jnp.int32),
        ]
        + [pltpu.VMEM((ch, d), jnp.float32) for _ in range(nb)]
        + [pltpu.SemaphoreType.DMA for _ in range(2 * nb)],
    )
    def gather_kernel(table_hbm, idx_hbm, out_hbm, idx_v, *rest):
        bufs = rest[:nb]
        gsems = rest[nb:2 * nb]
        wsems = rest[2 * nb:]
        wid = lax.axis_index("s") * 2 + lax.axis_index("c")
        bb = wid // w_per_batch
        col0 = k * cs + (wid % w_per_batch) * b_per_w
        base = wid * b_per_w
        pltpu.sync_copy(idx_hbm.at[bb, pl.ds(col0, b_per_w)], idx_v)

        def start_gather(c):
            cp = pltpu.make_async_copy(
                table_hbm.at[idx_v.at[pl.ds(c * ch, ch)]],
                bufs[c % nb],
                gsems[c % nb],
            )
            cp.start()
            return cp

        def start_wb(c):
            cp = pltpu.make_async_copy(
                bufs[c % nb],
                out_hbm.at[pl.ds(base + c * ch, ch)],
                wsems[c % nb],
            )
            cp.start()
            return cp

        gathers = [None] * n_ch
        wbs = [None] * n_ch
        for c in range(min(nb - 1, n_ch)):
            gathers[c] = start_gather(c)
        for c in range(n_ch):
            gathers[c].wait()
            wbs[c] = start_wb(c)
            j = c + nb - 1
            if j < n_ch:
                if c >= 1:
                    wbs[c - 1].wait()
                gathers[j] = start_gather(j)
        for c in range(max(0, n_ch - nb), n_ch):
            if wbs[c] is not None:
                wbs[c].wait()

    return gather_kernel(token_table, ids_2d)


def _ln_math(x, segtab_ref, gamma_ref, beta_ref, segf_row):
    s0 = segtab_ref[0, :]
    s1 = segtab_ref[1, :]
    segf = segf_row.reshape(-1, 1)
    x = x + s0[None, :] + segf * (s1 - s0)[None, :]
    mean = jnp.mean(x, axis=-1, keepdims=True)
    xc = x - mean
    var = jnp.mean(xc * xc, axis=-1, keepdims=True)
    return gamma_ref[...] * (xc * lax.rsqrt(var + EPS)) + beta_ref[...]


def _ln_body(g_ref, pos_ref, segf_ref, segtab_ref, gamma_ref, beta_ref, o_ref):
    x = g_ref[...] + pos_ref[...]
    o_ref[...] = _ln_math(x, segtab_ref, gamma_ref, beta_ref, segf_ref[0, 0, :])


def _ln_body_acc(g_ref, pos_ref, segf_ref, segtab_ref, gamma_ref, beta_ref,
                 prev_ref, o_ref):
    del prev_ref
    x = g_ref[...] + pos_ref[...]
    o_ref[...] = _ln_math(x, segtab_ref, gamma_ref, beta_ref, segf_ref[0, 0, :])


def _tc_ln_chunk(g_k, pos_table, segf, seg_table, gamma, beta, prev, off,
                 cs, batches, s_total, blk):
    d = g_k.shape[1]
    n_total = batches * s_total
    sb = s_total // blk
    ob = off // blk
    in_specs = [
        pl.BlockSpec((blk, d), lambda bb: (bb, 0)),
        pl.BlockSpec((blk, d), lambda bb: (ob, 0)),
        pl.BlockSpec((1, 1, blk), lambda bb: (bb, 0, ob)),
        pl.BlockSpec((2, d), lambda bb: (0, 0)),
        pl.BlockSpec((1, d), lambda bb: (0, 0)),
        pl.BlockSpec((1, d), lambda bb: (0, 0)),
    ]
    args = [g_k, pos_table, segf, seg_table, gamma, beta]
    if prev is None:
        body = _ln_body
        aliases = {}
    else:
        body = _ln_body_acc
        in_specs = in_specs + [pl.BlockSpec(memory_space=pltpu.MemorySpace.HBM)]
        args = args + [prev]
        aliases = {6: 0}
    return pl.pallas_call(
        body,
        grid=(batches,),
        in_specs=in_specs,
        out_specs=pl.BlockSpec((blk, d), lambda bb: (bb * sb + ob, 0)),
        out_shape=jax.ShapeDtypeStruct((n_total, d), jnp.float32),
        input_output_aliases=aliases,
    )(*args)


def kernel(input_ids, segment_ids, token_table, pos_table, seg_table, gamma, beta):
    b, s = input_ids.shape
    cs = s // _NCHUNKS
    ids32 = input_ids.astype(jnp.int32)
    segf = segment_ids.astype(jnp.float32).reshape(b, 1, s)
    gamma2 = gamma.reshape(1, -1)
    beta2 = beta.reshape(1, -1)

    gathered = [
        _sc_gather_chunk(token_table, ids32, k, cs, ch=16, nb=7)
        for k in range(_NCHUNKS)
    ]

    out = None
    for k in range(_NCHUNKS):
        out = _tc_ln_chunk(
            gathered[k], pos_table, segf, seg_table, gamma2, beta2,
            out, k * cs, cs, b, s, blk=cs,
        )
    return out.reshape(b, s, D_MODEL)
